# baseline (speedup 1.0000x reference)
import functools

import jax
import jax.numpy as jnp
from jax import lax
from jax.experimental import pallas as pl
from jax.experimental.pallas import tpu as pltpu
from jax.experimental.pallas import tpu_sc as plsc

B, L, D = 4096, 200, 64
K1, K2, K3 = 5, 21, 85
KTOT = K1 + K2 + K3
KPAD = 112
NC, NS, LANES = 2, 16, 16
NW = NC * NS
ROWS_PER_W = B // NW
GRP = 64
NGRP = ROWS_PER_W // GRP
ROW_ELEMS = L * D
NCHUNK = 13


def _sc_embed(t1, t2, t3, tabflat, fcv, fcbv):
    mesh = plsc.VectorSubcoreMesh(core_axis_name="c", subcore_axis_name="s")

    @functools.partial(
        pl.kernel,
        out_type=jax.ShapeDtypeStruct((B, ROW_ELEMS), jnp.float32),
        mesh=mesh,
        scratch_types=[
            pltpu.VMEM((KPAD * D,), jnp.float32),
            pltpu.VMEM((128,), jnp.float32),
            pltpu.VMEM((LANES,), jnp.float32),
            pltpu.VMEM((128,), jnp.float32),
            pltpu.VMEM((GRP, L), jnp.int32),
            pltpu.VMEM((GRP, L), jnp.int32),
            pltpu.VMEM((GRP, L), jnp.int32),
            pltpu.VMEM((L,), jnp.float32),
            pltpu.VMEM((L,), jnp.float32),
            pltpu.VMEM((L,), jnp.float32),
            pltpu.VMEM((ROW_ELEMS,), jnp.float32),
            pltpu.VMEM((ROW_ELEMS,), jnp.float32),
            pltpu.SemaphoreType.DMA,
            pltpu.SemaphoreType.DMA,
        ],
        compiler_params=pltpu.CompilerParams(needs_layout_passes=False),
    )
    def k(t1h, t2h, t3h, tabh, fch, fcbh, outh,
          tab_v, fc_v, fcb_v, lt_v, i1_v, i2_v, i3_v, w1_v, w2_v, w3_v,
          ob0_v, ob1_v, sem0, sem1):
        wid = lax.axis_index("s") * NC + lax.axis_index("c")

        pltpu.sync_copy(tabh, tab_v)
        pltpu.sync_copy(fch, fc_v)
        pltpu.sync_copy(fcbh, fcb_v)

        iota = lax.iota(jnp.int32, LANES)
        iota_d = iota * D
        tail_keep = iota >= 8
        zero_v = jnp.zeros((LANES,), jnp.float32)

        def lt_chunk(kc, _):
            rows = iota + kc * LANES
            def dbody(d, carry):
                acc, idx = carry
                fd = plsc.load_gather(fc_v, [lax.broadcast(d, (LANES,))])
                val = plsc.load_gather(tab_v, [idx])
                return acc + val * fd, idx + 1
            acc0 = fcb_v[...]
            acc, _ = lax.fori_loop(0, D, dbody, (acc0, rows * D))
            lt_v[pl.ds(kc * LANES, LANES)] = acc
            return 0
        lax.fori_loop(0, KPAD // LANES, lt_chunk, 0)

        def softmax_weights(idx_ref, w_ref, l_off, r):
            lg = []
            for v in range(NCHUNK):
                off = 184 if v == NCHUNK - 1 else 16 * v
                tv = idx_ref[r, pl.ds(off, 16)]
                if l_off:
                    tv = tv + l_off
                lg.append(plsc.load_gather(lt_v, [tv]))
            m = lg[0]
            for v in range(1, NCHUNK):
                m = jnp.maximum(m, lg[v])
            mb = lax.broadcast(jnp.max(m), (LANES,))
            es = [jnp.exp(x - mb) for x in lg]
            s = es[0]
            for v in range(1, NCHUNK - 1):
                s = s + es[v]
            s = s + jnp.where(tail_keep, es[NCHUNK - 1], zero_v)
            rec = 1.0 / lax.broadcast(jnp.sum(s), (LANES,))
            for v in range(NCHUNK):
                off = 184 if v == NCHUNK - 1 else 16 * v
                w_ref[pl.ds(off, 16)] = es[v] * rec
            return

        row_base = wid * ROWS_PER_W


        def fuse_row(r, ob_v):
            softmax_weights(i1_v, w1_v, 0, r)
            softmax_weights(i2_v, w2_v, K1, r)
            softmax_weights(i3_v, w3_v, K1 + K2, r)

            def chunk_body(c):
                off = jnp.minimum(c * 16, 184)
                t1v = i1_v[r, pl.ds(off, 16)]
                t2v = i2_v[r, pl.ds(off, 16)]
                t3v = i3_v[r, pl.ds(off, 16)]
                w1v = w1_v[pl.ds(off, 16)]
                w2v = w2_v[pl.ds(off, 16)]
                w3v = w3_v[pl.ds(off, 16)]
                b1 = t1v * D
                b2 = t2v * D + K1 * D
                b3 = t3v * D + (K1 + K2) * D
                obb = iota_d + lax.broadcast(off * D, (LANES,))

                def dbody(_d, dvec):
                    g1 = plsc.load_gather(tab_v, [b1 + dvec])
                    g2 = plsc.load_gather(tab_v, [b2 + dvec])
                    g3 = plsc.load_gather(tab_v, [b3 + dvec])
                    val = w1v * g1 + w2v * g2 + w3v * g3
                    plsc.store_scatter(ob_v, [obb + dvec], val)
                    return (dvec + 1) & (D - 1)

                plsc.parallel_loop(0, D, carry=iota, unroll=16)(dbody)

            plsc.parallel_loop(0, NCHUNK)(chunk_body)

        def grp_body(g, _):
            row0 = row_base + g * GRP
            pltpu.sync_copy(t1h.at[pl.ds(row0, GRP)], i1_v)
            pltpu.sync_copy(t2h.at[pl.ds(row0, GRP)], i2_v)
            pltpu.sync_copy(t3h.at[pl.ds(row0, GRP)], i3_v)

            fuse_row(0, ob0_v)
            pltpu.async_copy(ob0_v, outh.at[row0], sem0)
            fuse_row(1, ob1_v)
            pltpu.async_copy(ob1_v, outh.at[row0 + 1], sem1)

            def pair_body(p, _):
                r = 2 * p
                pltpu.make_async_copy(ob0_v, outh.at[row0 + r], sem0).wait()
                fuse_row(r, ob0_v)
                pltpu.async_copy(ob0_v, outh.at[row0 + r], sem0)
                pltpu.make_async_copy(
                    ob1_v, outh.at[row0 + r + 1], sem1).wait()
                fuse_row(r + 1, ob1_v)
                pltpu.async_copy(ob1_v, outh.at[row0 + r + 1], sem1)
                return 0

            lax.fori_loop(1, GRP // 2, pair_body, 0)
            pltpu.make_async_copy(ob0_v, outh.at[row0], sem0).wait()
            pltpu.make_async_copy(ob1_v, outh.at[row0 + 1], sem1).wait()
            return 0

        lax.fori_loop(0, ROWS_PER_W // GRP, grp_body, 0)

    return k(t1, t2, t3, tabflat, fcv, fcbv)


def kernel(t1, t2, t3, W1, W2, W3, fc_w, fc_b):
    tab = jnp.concatenate([W1, W2, W3], axis=0)
    tab = jnp.pad(tab, ((0, KPAD - KTOT), (0, 0)))
    tabflat = tab.reshape(-1)
    fcv = jnp.pad(fc_w.reshape(D), (0, 128 - D))
    fcbv = jnp.full((LANES,), fc_b[0], jnp.float32)
    out = _sc_embed(t1.astype(jnp.int32), t2.astype(jnp.int32),
                    t3.astype(jnp.int32), tabflat, fcv, fcbv)
    return out.reshape(B, L, D)


# speedup vs baseline: 1.5618x; 1.5618x over previous
import functools

import jax
import jax.numpy as jnp
from jax import lax
from jax.experimental import pallas as pl
from jax.experimental.pallas import tpu as pltpu
from jax.experimental.pallas import tpu_sc as plsc

B, L, D = 4096, 200, 64
K1, K2, K3 = 5, 21, 85
KTOT = K1 + K2 + K3
KPAD = 112
NC, NS, LANES = 2, 16, 16
NW = NC * NS
ROWS_PER_W = B // NW
GRP = 64
NGRP = ROWS_PER_W // GRP
ROW_ELEMS = L * D
NCHUNK = 13


def _sc_embed(t1, t2, t3, tabflat, fcv, fcbv):
    mesh = plsc.VectorSubcoreMesh(core_axis_name="c", subcore_axis_name="s")

    @functools.partial(
        pl.kernel,
        out_type=jax.ShapeDtypeStruct((B, ROW_ELEMS), jnp.float32),
        mesh=mesh,
        scratch_types=[
            pltpu.VMEM((KPAD * D,), jnp.float32),
            pltpu.VMEM((128,), jnp.float32),
            pltpu.VMEM((LANES,), jnp.float32),
            pltpu.VMEM((128,), jnp.float32),
            pltpu.VMEM((GRP, L), jnp.int32),
            pltpu.VMEM((GRP, L), jnp.int32),
            pltpu.VMEM((GRP, L), jnp.int32),
            pltpu.VMEM((L,), jnp.float32),
            pltpu.VMEM((L,), jnp.float32),
            pltpu.VMEM((L,), jnp.float32),
            pltpu.VMEM((ROW_ELEMS,), jnp.float32),
            pltpu.VMEM((ROW_ELEMS,), jnp.float32),
            pltpu.SemaphoreType.DMA,
            pltpu.SemaphoreType.DMA,
        ],
        compiler_params=pltpu.CompilerParams(needs_layout_passes=False),
    )
    def k(t1h, t2h, t3h, tabh, fch, fcbh, outh,
          tab_v, fc_v, fcb_v, lt_v, i1_v, i2_v, i3_v, w1_v, w2_v, w3_v,
          ob0_v, ob1_v, sem0, sem1):
        wid = lax.axis_index("s") * NC + lax.axis_index("c")

        pltpu.sync_copy(tabh, tab_v)
        pltpu.sync_copy(fch, fc_v)
        pltpu.sync_copy(fcbh, fcb_v)

        iota = lax.iota(jnp.int32, LANES)
        iota_d = iota * D
        tail_keep = iota >= 8
        zero_v = jnp.zeros((LANES,), jnp.float32)

        def lt_chunk(kc, _):
            rows = iota + kc * LANES
            def dbody(d, carry):
                acc, idx = carry
                fd = plsc.load_gather(fc_v, [lax.broadcast(d, (LANES,))])
                val = plsc.load_gather(tab_v, [idx])
                return acc + val * fd, idx + 1
            acc0 = fcb_v[...]
            acc, _ = lax.fori_loop(0, D, dbody, (acc0, rows * D))
            lt_v[pl.ds(kc * LANES, LANES)] = acc
            return 0
        lax.fori_loop(0, KPAD // LANES, lt_chunk, 0)

        def softmax_weights(idx_ref, w_ref, l_off, r):
            lg = []
            for v in range(NCHUNK):
                off = 184 if v == NCHUNK - 1 else 16 * v
                tv = idx_ref[r, pl.ds(off, 16)]
                if l_off:
                    tv = tv + l_off
                lg.append(plsc.load_gather(lt_v, [tv]))
            m = lg[0]
            for v in range(1, NCHUNK):
                m = jnp.maximum(m, lg[v])
            mb = lax.broadcast(jnp.max(m), (LANES,))
            es = [jnp.exp(x - mb) for x in lg]
            s = es[0]
            for v in range(1, NCHUNK - 1):
                s = s + es[v]
            s = s + jnp.where(tail_keep, es[NCHUNK - 1], zero_v)
            rec = 1.0 / lax.broadcast(jnp.sum(s), (LANES,))
            for v in range(NCHUNK):
                off = 184 if v == NCHUNK - 1 else 16 * v
                w_ref[pl.ds(off, 16)] = es[v] * rec
            return

        row_base = wid * ROWS_PER_W


        def fuse_row(r, ob_v):
            softmax_weights(i1_v, w1_v, 0, r)
            softmax_weights(i2_v, w2_v, K1, r)
            softmax_weights(i3_v, w3_v, K1 + K2, r)

            def chunk_body(c):
                off = jnp.minimum(c * 16, 184)
                t1v = i1_v[r, pl.ds(off, 16)]
                t2v = i2_v[r, pl.ds(off, 16)]
                t3v = i3_v[r, pl.ds(off, 16)]
                w1v = w1_v[pl.ds(off, 16)]
                w2v = w2_v[pl.ds(off, 16)]
                w3v = w3_v[pl.ds(off, 16)]
                b1 = t1v * D
                b2 = t2v * D + K1 * D
                b3 = t3v * D + (K1 + K2) * D
                obb = iota_d + lax.broadcast(off * D, (LANES,))

                def dbody(_d, dvec):
                    g1 = plsc.load_gather(tab_v, [b1 + dvec])
                    g2 = plsc.load_gather(tab_v, [b2 + dvec])
                    g3 = plsc.load_gather(tab_v, [b3 + dvec])
                    val = w1v * g1 + w2v * g2 + w3v * g3
                    plsc.store_scatter(ob_v, [obb + dvec], val)
                    return (dvec + 1) & (D - 1)

                plsc.parallel_loop(0, D, carry=iota, unroll=4)(dbody)

            plsc.parallel_loop(0, NCHUNK)(chunk_body)

        def grp_body(g, _):
            row0 = row_base + g * GRP
            pltpu.sync_copy(t1h.at[pl.ds(row0, GRP)], i1_v)
            pltpu.sync_copy(t2h.at[pl.ds(row0, GRP)], i2_v)
            pltpu.sync_copy(t3h.at[pl.ds(row0, GRP)], i3_v)

            fuse_row(0, ob0_v)
            pltpu.async_copy(ob0_v, outh.at[row0], sem0)
            fuse_row(1, ob1_v)
            pltpu.async_copy(ob1_v, outh.at[row0 + 1], sem1)

            def pair_body(p, _):
                r = 2 * p
                pltpu.make_async_copy(ob0_v, outh.at[row0 + r], sem0).wait()
                fuse_row(r, ob0_v)
                pltpu.async_copy(ob0_v, outh.at[row0 + r], sem0)
                pltpu.make_async_copy(
                    ob1_v, outh.at[row0 + r + 1], sem1).wait()
                fuse_row(r + 1, ob1_v)
                pltpu.async_copy(ob1_v, outh.at[row0 + r + 1], sem1)
                return 0

            lax.fori_loop(1, GRP // 2, pair_body, 0)
            pltpu.make_async_copy(ob0_v, outh.at[row0], sem0).wait()
            pltpu.make_async_copy(ob1_v, outh.at[row0 + 1], sem1).wait()
            return 0

        lax.fori_loop(0, ROWS_PER_W // GRP, grp_body, 0)

    return k(t1, t2, t3, tabflat, fcv, fcbv)


def kernel(t1, t2, t3, W1, W2, W3, fc_w, fc_b):
    tab = jnp.concatenate([W1, W2, W3], axis=0)
    tab = jnp.pad(tab, ((0, KPAD - KTOT), (0, 0)))
    tabflat = tab.reshape(-1)
    fcv = jnp.pad(fc_w.reshape(D), (0, 128 - D))
    fcbv = jnp.full((LANES,), fc_b[0], jnp.float32)
    out = _sc_embed(t1.astype(jnp.int32), t2.astype(jnp.int32),
                    t3.astype(jnp.int32), tabflat, fcv, fcbv)
    return out.reshape(B, L, D)
